# baseline (speedup 1.0000x reference)
import functools

import jax
import jax.numpy as jnp
from jax import lax
from jax.experimental import pallas as pl
from jax.experimental.pallas import tpu as pltpu
from jax.experimental.pallas import tpu_sc as plsc

D_VOCAB = 1000000
D_EMB = 64
B_TOK = 1024 * 200

_TC_CHUNK = 8192

NC, NS = 2, 16
NW = NC * NS
ROWS_PER_W = B_TOK // NW
IDX_MINOR = 128
CHUNKS_PER_W = ROWS_PER_W // IDX_MINOR


def _transpose_body(w_ref, o_ref):
    o_ref[...] = w_ref[...].T


def _transpose_table(W_E):
    return pl.pallas_call(
        _transpose_body,
        grid=(pl.cdiv(D_VOCAB, _TC_CHUNK),),
        in_specs=[pl.BlockSpec((D_EMB, _TC_CHUNK), lambda i: (0, i))],
        out_specs=pl.BlockSpec((_TC_CHUNK, D_EMB), lambda i: (i, 0)),
        out_shape=jax.ShapeDtypeStruct((D_VOCAB, D_EMB), jnp.float32),
    )(W_E)


def _make_gather():
    mesh = plsc.VectorSubcoreMesh(core_axis_name="c", subcore_axis_name="s")

    @functools.partial(
        pl.kernel,
        mesh=mesh,
        out_type=jax.ShapeDtypeStruct((B_TOK, D_EMB), jnp.float32),
        compiler_params=pltpu.CompilerParams(use_tc_tiling_on_sc=False),
        scratch_types=[
            pltpu.VMEM((CHUNKS_PER_W, IDX_MINOR), jnp.int32),
            pltpu.VMEM((IDX_MINOR, D_EMB), jnp.float32),
            pltpu.SemaphoreType.DMA,
        ],
    )
    def gather(table_hbm, idx_hbm, out_hbm, idx_v, rows_v, sem):
        wid = lax.axis_index("s") * NC + lax.axis_index("c")
        base_chunk = wid * CHUNKS_PER_W
        pltpu.sync_copy(idx_hbm.at[wid], idx_v)

        def body(c, _):
            pltpu.async_copy(table_hbm.at[idx_v.at[c]], rows_v, sem).wait()
            pltpu.sync_copy(
                rows_v, out_hbm.at[pl.ds((base_chunk + c) * IDX_MINOR, IDX_MINOR)]
            )
            return _

        lax.fori_loop(0, CHUNKS_PER_W, body, None)

    return gather


_gather = _make_gather()


def kernel(x, W_E):
    table = _transpose_table(W_E)
    idx = x.reshape(NW, CHUNKS_PER_W, IDX_MINOR).astype(jnp.int32)
    out = _gather(table, idx)
    return out.reshape(1024, 200, D_EMB)
